# baseline (speedup 1.0000x reference)
import functools

import jax
import jax.numpy as jnp
from jax import lax
from jax.experimental import pallas as pl
from jax.experimental.pallas import tpu as pltpu
from jax.experimental.pallas import tpu_sc as plsc

NC = 2
NS = 16
NW = NC * NS
CHUNK = 128
RPT = 632


def _fill_2d(buf, rows, cols, value):
    per_row = cols // 16

    def body(t, carry):
        r = t // per_row
        k = t % per_row
        buf[r, pl.ds(k * 16, 16)] = jnp.full((16,), value, jnp.float32)
        return carry

    lax.fori_loop(0, rows * per_row, body, 0)


def _zero_acc_slice(zbuf, acc, sid):
    base = sid * RPT
    n_full = RPT // CHUNK
    rem = RPT % CHUNK
    for k in range(n_full):
        pltpu.sync_copy(zbuf, acc.at[pl.ds(base + k * CHUNK, CHUNK)])
    if rem:
        pltpu.sync_copy(zbuf.at[pl.ds(0, rem)],
                        acc.at[pl.ds(base + n_full * CHUNK, rem)])


def _make_deg_kernel(n_edges):
    n_iters = n_edges // (CHUNK * NW)
    assert n_iters % 2 == 0
    n_pad = NS * RPT
    DW = 128

    mesh = plsc.VectorSubcoreMesh(core_axis_name="c", subcore_axis_name="s",
                                  num_cores=NC, num_subcores=NS)

    @functools.partial(
        pl.kernel,
        out_type=jax.ShapeDtypeStruct((NC, NS, RPT, DW), jnp.float32),
        mesh=mesh,
        scratch_types=[
            pltpu.VMEM((CHUNK,), jnp.int32),
            pltpu.VMEM((CHUNK,), jnp.int32),
            pltpu.VMEM((CHUNK, DW), jnp.float32),
            pltpu.VMEM((CHUNK, DW), jnp.float32),
            pltpu.SemaphoreType.DMA,
            pltpu.SemaphoreType.DMA,
            pltpu.VMEM_SHARED((n_pad, DW), jnp.float32),
        ],
    )
    def deg_kernel(dst_hbm, out_hbm, dst0, dst1, ones_v, zbuf, semi0, semi1,
                   acc):
        cid = lax.axis_index("c")
        sid = lax.axis_index("s")
        wid = sid * NC + cid

        def d_src(i):
            return dst_hbm.at[pl.ds((wid * n_iters + i) * CHUNK, CHUNK)]

        pltpu.sync_copy(d_src(0), dst0)
        pltpu.async_copy(d_src(1), dst1, semi1)
        _fill_2d(ones_v, CHUNK, DW, 1.0)
        _fill_2d(zbuf, CHUNK, DW, 0.0)
        _zero_acc_slice(zbuf, acc, sid)
        plsc.subcore_barrier()

        last = n_iters - 1

        def body(j, carry):
            i0 = 2 * j
            i1 = 2 * j + 1
            i2 = jnp.minimum(i0 + 2, last)
            i3 = jnp.minimum(i1 + 2, last)
            pltpu.sync_copy(ones_v, acc.at[dst0], add=True)
            pltpu.make_async_copy(d_src(i1), dst1, semi1).wait()
            pltpu.async_copy(d_src(i2), dst0, semi0)
            pltpu.sync_copy(ones_v, acc.at[dst1], add=True)
            pltpu.make_async_copy(d_src(i2), dst0, semi0).wait()
            pltpu.async_copy(d_src(i3), dst1, semi1)
            return carry

        lax.fori_loop(0, n_iters // 2, body, 0)
        pltpu.make_async_copy(d_src(last), dst1, semi1).wait()
        plsc.subcore_barrier()
        pltpu.sync_copy(acc.at[pl.ds(sid * RPT, RPT)], out_hbm.at[cid, sid])

    return deg_kernel


def _make_agg_kernel(n_edges, d):
    n_iters = n_edges // (CHUNK * NW)
    assert n_iters % 2 == 0
    n_pad = NS * RPT

    mesh = plsc.VectorSubcoreMesh(core_axis_name="c", subcore_axis_name="s",
                                  num_cores=NC, num_subcores=NS)

    @functools.partial(
        pl.kernel,
        out_type=jax.ShapeDtypeStruct((NC, NS, RPT, d), jnp.float32),
        mesh=mesh,
        scratch_types=[
            pltpu.VMEM((CHUNK,), jnp.int32),
            pltpu.VMEM((CHUNK,), jnp.int32),
            pltpu.VMEM((CHUNK,), jnp.int32),
            pltpu.VMEM((CHUNK,), jnp.int32),
            pltpu.VMEM((CHUNK, d), jnp.float32),
            pltpu.VMEM((CHUNK, d), jnp.float32),
            pltpu.SemaphoreType.DMA,
            pltpu.SemaphoreType.DMA,
            pltpu.SemaphoreType.DMA,
            pltpu.SemaphoreType.DMA,
            pltpu.VMEM_SHARED((n_pad, d), jnp.float32),
        ],
    )
    def agg_kernel(y_hbm, src_hbm, dst_hbm, out_hbm,
                   src0, src1, dst0, dst1, rows0, rows1,
                   semi0, semi1, semg0, semg1, acc):
        cid = lax.axis_index("c")
        sid = lax.axis_index("s")
        wid = sid * NC + cid

        def s_src(i):
            return src_hbm.at[pl.ds((wid * n_iters + i) * CHUNK, CHUNK)]

        def d_src(i):
            return dst_hbm.at[pl.ds((wid * n_iters + i) * CHUNK, CHUNK)]

        pltpu.sync_copy(s_src(0), src0)
        pltpu.sync_copy(d_src(0), dst0)
        pltpu.async_copy(y_hbm.at[src0], rows0, semg0)
        pltpu.async_copy(s_src(1), src1, semi1)
        pltpu.async_copy(d_src(1), dst1, semi1)
        _fill_2d(rows1, CHUNK, d, 0.0)
        _zero_acc_slice(rows1, acc, sid)
        plsc.subcore_barrier()

        last = n_iters - 1

        def body(j, carry):
            i1 = 2 * j + 1
            i2 = jnp.minimum(i1 + 1, last)
            i3 = jnp.minimum(i1 + 2, last)
            pltpu.make_async_copy(s_src(i1), src1, semi1).wait()
            pltpu.make_async_copy(d_src(i1), dst1, semi1).wait()
            pltpu.async_copy(y_hbm.at[src1], rows1, semg1)
            pltpu.make_async_copy(y_hbm.at[src0], rows0, semg0).wait()
            pltpu.sync_copy(rows0, acc.at[dst0], add=True)
            pltpu.async_copy(s_src(i2), src0, semi0)
            pltpu.async_copy(d_src(i2), dst0, semi0)
            pltpu.make_async_copy(s_src(i2), src0, semi0).wait()
            pltpu.make_async_copy(d_src(i2), dst0, semi0).wait()
            pltpu.async_copy(y_hbm.at[src0], rows0, semg0)
            pltpu.make_async_copy(y_hbm.at[src1], rows1, semg1).wait()
            pltpu.sync_copy(rows1, acc.at[dst1], add=True)
            pltpu.async_copy(s_src(i3), src1, semi1)
            pltpu.async_copy(d_src(i3), dst1, semi1)
            return carry

        lax.fori_loop(0, n_iters // 2, body, 0)
        pltpu.make_async_copy(y_hbm.at[src0], rows0, semg0).wait()
        pltpu.make_async_copy(s_src(last), src1, semi1).wait()
        pltpu.make_async_copy(d_src(last), dst1, semi1).wait()
        plsc.subcore_barrier()
        pltpu.sync_copy(acc.at[pl.ds(sid * RPT, RPT)], out_hbm.at[cid, sid])

    return agg_kernel


_BR = 1000


def _tc_first(x, w, dinv2):
    n, d_in = x.shape
    d_out = w.shape[1]

    def body(x_ref, w_ref, dinv_ref, y_ref):
        xw = jnp.dot(x_ref[...], w_ref[...],
                     preferred_element_type=jnp.float32)
        y_ref[...] = xw * dinv_ref[...]

    return pl.pallas_call(
        body,
        grid=(n // _BR,),
        in_specs=[
            pl.BlockSpec((_BR, d_in), lambda i: (i, 0)),
            pl.BlockSpec((d_in, d_out), lambda i: (0, 0)),
            pl.BlockSpec((_BR, 1), lambda i: (i, 0)),
        ],
        out_specs=pl.BlockSpec((_BR, d_out), lambda i: (i, 0)),
        out_shape=jax.ShapeDtypeStruct((n, d_out), jnp.float32),
    )(x, w, dinv2)


def _tc_mid(p0, p1, y_prev, dinv2, b, w):
    n, d = y_prev.shape
    d_out = w.shape[1]

    def body(p0_ref, p1_ref, y_ref, dinv_ref, b_ref, w_ref, out_ref):
        agg = p0_ref[...] + p1_ref[...] + y_ref[...]
        h = jnp.maximum(agg * dinv_ref[...] + b_ref[...], 0.0)
        hw = jnp.dot(h, w_ref[...], preferred_element_type=jnp.float32)
        out_ref[...] = hw * dinv_ref[...]

    return pl.pallas_call(
        body,
        grid=(n // _BR,),
        in_specs=[
            pl.BlockSpec((_BR, d), lambda i: (i, 0)),
            pl.BlockSpec((_BR, d), lambda i: (i, 0)),
            pl.BlockSpec((_BR, d), lambda i: (i, 0)),
            pl.BlockSpec((_BR, 1), lambda i: (i, 0)),
            pl.BlockSpec((1, d), lambda i: (0, 0)),
            pl.BlockSpec((d, d_out), lambda i: (0, 0)),
        ],
        out_specs=pl.BlockSpec((_BR, d_out), lambda i: (i, 0)),
        out_shape=jax.ShapeDtypeStruct((n, d_out), jnp.float32),
    )(p0, p1, y_prev, dinv2, b, w)


def _tc_scale(p0, p1, y_prev, dinv2, b):
    n, d = y_prev.shape

    def body(p0_ref, p1_ref, y_ref, dinv_ref, b_ref, out_ref):
        agg = p0_ref[...] + p1_ref[...] + y_ref[...]
        h = jnp.maximum(agg * dinv_ref[...] + b_ref[...], 0.0)
        out_ref[...] = h * dinv_ref[...]

    return pl.pallas_call(
        body,
        grid=(n // _BR,),
        in_specs=[
            pl.BlockSpec((_BR, d), lambda i: (i, 0)),
            pl.BlockSpec((_BR, d), lambda i: (i, 0)),
            pl.BlockSpec((_BR, d), lambda i: (i, 0)),
            pl.BlockSpec((_BR, 1), lambda i: (i, 0)),
            pl.BlockSpec((1, d), lambda i: (0, 0)),
        ],
        out_specs=pl.BlockSpec((_BR, d), lambda i: (i, 0)),
        out_shape=jax.ShapeDtypeStruct((n, d), jnp.float32),
    )(p0, p1, y_prev, dinv2, b)


def _tc_final(p0, p1, t_prev, dinv2, w, b):
    n, d = t_prev.shape
    d_out = w.shape[1]

    def body(p0_ref, p1_ref, t_ref, dinv_ref, w_ref, b_ref, out_ref):
        agg = (p0_ref[...] + p1_ref[...] + t_ref[...]) * dinv_ref[...]
        z = jnp.dot(agg, w_ref[...],
                    preferred_element_type=jnp.float32) + b_ref[...]
        m = jnp.max(z, axis=1, keepdims=True)
        e = jnp.exp(z - m)
        s = jnp.sum(e, axis=1, keepdims=True)
        out_ref[...] = z - m - jnp.log(s)

    return pl.pallas_call(
        body,
        grid=(n // _BR,),
        in_specs=[
            pl.BlockSpec((_BR, d), lambda i: (i, 0)),
            pl.BlockSpec((_BR, d), lambda i: (i, 0)),
            pl.BlockSpec((_BR, d), lambda i: (i, 0)),
            pl.BlockSpec((_BR, 1), lambda i: (i, 0)),
            pl.BlockSpec((d, d_out), lambda i: (0, 0)),
            pl.BlockSpec((1, d_out), lambda i: (0, 0)),
        ],
        out_specs=pl.BlockSpec((_BR, d_out), lambda i: (i, 0)),
        out_shape=jax.ShapeDtypeStruct((n, d_out), jnp.float32),
    )(p0, p1, t_prev, dinv2, w, b)


def kernel(x, edge_index, W_in, b_in, W_h, b_h, W_out, b_out):
    n, d_in = x.shape
    n_edges = edge_index.shape[1]
    d_h = W_h.shape[0]
    d_out = W_out.shape[1]
    n_pad = NS * RPT

    ei = edge_index.astype(jnp.int32)
    e_blk = 2 * CHUNK * NW
    e_pad = ((n_edges + e_blk - 1) // e_blk) * e_blk
    src = jnp.concatenate(
        [ei[0], jnp.zeros((e_pad - n_edges,), jnp.int32)])
    dst = jnp.concatenate(
        [ei[1], jnp.full((e_pad - n_edges,), n_pad - 1, jnp.int32)])
    degp = _make_deg_kernel(e_pad)(dst)
    degp = degp.reshape(NC, n_pad, 128)
    deg = degp[0, :n, 0] + degp[1, :n, 0] + 1.0
    dinv2 = lax.rsqrt(deg)[:, None]

    agg_h = _make_agg_kernel(e_pad, d_h)

    y1 = _tc_first(x, W_in, dinv2)
    p = agg_h(y1, src, dst).reshape(NC, n_pad, d_h)[:, :n]
    y2 = _tc_mid(p[0], p[1], y1, dinv2, b_in.reshape(1, -1), W_h)
    p = agg_h(y2, src, dst).reshape(NC, n_pad, d_h)[:, :n]
    t3 = _tc_scale(p[0], p[1], y2, dinv2, b_h.reshape(1, -1))
    p = agg_h(t3, src, dst).reshape(NC, n_pad, d_h)[:, :n]
    return _tc_final(p[0], p[1], t3, dinv2, W_out, b_out.reshape(1, -1))


# speedup vs baseline: 1.6180x; 1.6180x over previous
import functools

import jax
import jax.numpy as jnp
from jax import lax
from jax.experimental import pallas as pl
from jax.experimental.pallas import tpu as pltpu
from jax.experimental.pallas import tpu_sc as plsc

NC = 2
NS = 16
NW = NC * NS
CHUNK = 64
RPT = 632


def _fill_2d(buf, rows, cols, value):
    per_row = cols // 16

    def body(t, carry):
        r = t // per_row
        k = t % per_row
        buf[r, pl.ds(k * 16, 16)] = jnp.full((16,), value, jnp.float32)
        return carry

    lax.fori_loop(0, rows * per_row, body, 0)


def _zero_acc_slice(zbuf, acc, sid):
    base = sid * RPT
    n_full = RPT // CHUNK
    rem = RPT % CHUNK
    for k in range(n_full):
        pltpu.sync_copy(zbuf, acc.at[pl.ds(base + k * CHUNK, CHUNK)])
    if rem:
        pltpu.sync_copy(zbuf.at[pl.ds(0, rem)],
                        acc.at[pl.ds(base + n_full * CHUNK, rem)])


def _make_deg_kernel(n_edges):
    n_iters = n_edges // (CHUNK * NW)
    n_pad = NS * RPT
    DW = 128

    mesh = plsc.VectorSubcoreMesh(core_axis_name="c", subcore_axis_name="s",
                                  num_cores=NC, num_subcores=NS)

    @functools.partial(
        pl.kernel,
        out_type=jax.ShapeDtypeStruct((NC, NS, RPT, DW), jnp.float32),
        mesh=mesh,
        scratch_types=[
            pltpu.VMEM((n_iters * CHUNK,), jnp.int32),
            pltpu.VMEM((CHUNK, DW), jnp.float32),
            pltpu.VMEM((CHUNK, DW), jnp.float32),
            pltpu.VMEM_SHARED((n_pad, DW), jnp.float32),
        ],
    )
    def deg_kernel(dst_hbm, out_hbm, dst_i, ones_v, zbuf, acc):
        cid = lax.axis_index("c")
        sid = lax.axis_index("s")
        wid = sid * NC + cid
        epw = n_iters * CHUNK
        pltpu.sync_copy(dst_hbm.at[pl.ds(wid * epw, epw)], dst_i)
        _fill_2d(ones_v, CHUNK, DW, 1.0)
        _fill_2d(zbuf, CHUNK, DW, 0.0)
        _zero_acc_slice(zbuf, acc, sid)
        plsc.subcore_barrier()

        def body(i, carry):
            pltpu.sync_copy(ones_v, acc.at[dst_i.at[pl.ds(i * CHUNK, CHUNK)]],
                            add=True)
            return carry

        lax.fori_loop(0, n_iters, body, 0)
        plsc.subcore_barrier()
        pltpu.sync_copy(acc.at[pl.ds(sid * RPT, RPT)], out_hbm.at[cid, sid])

    return deg_kernel


def _make_agg_kernel(n_edges, d):
    n_iters = n_edges // (CHUNK * NW)
    assert n_iters % 2 == 0
    n_pad = NS * RPT

    mesh = plsc.VectorSubcoreMesh(core_axis_name="c", subcore_axis_name="s",
                                  num_cores=NC, num_subcores=NS)

    @functools.partial(
        pl.kernel,
        out_type=jax.ShapeDtypeStruct((NC, NS, RPT, d), jnp.float32),
        mesh=mesh,
        scratch_types=[
            pltpu.VMEM((n_iters * CHUNK,), jnp.int32),
            pltpu.VMEM((n_iters * CHUNK,), jnp.int32),
            pltpu.VMEM((CHUNK, d), jnp.float32),
            pltpu.VMEM((CHUNK, d), jnp.float32),
            pltpu.SemaphoreType.DMA,
            pltpu.SemaphoreType.DMA,
            pltpu.VMEM_SHARED((n_pad, d), jnp.float32),
        ],
    )
    def agg_kernel(y_hbm, src_hbm, dst_hbm, out_hbm,
                   src_i, dst_i, rows0, rows1, semg0, semg1, acc):
        cid = lax.axis_index("c")
        sid = lax.axis_index("s")
        wid = sid * NC + cid
        epw = n_iters * CHUNK
        pltpu.sync_copy(src_hbm.at[pl.ds(wid * epw, epw)], src_i)
        pltpu.sync_copy(dst_hbm.at[pl.ds(wid * epw, epw)], dst_i)
        _fill_2d(rows0, CHUNK, d, 0.0)
        _zero_acc_slice(rows0, acc, sid)
        plsc.subcore_barrier()

        def s_at(i):
            return src_i.at[pl.ds(i * CHUNK, CHUNK)]

        def d_at(i):
            return dst_i.at[pl.ds(i * CHUNK, CHUNK)]

        last = n_iters - 1
        pltpu.async_copy(y_hbm.at[s_at(0)], rows0, semg0)

        def body(j, carry):
            i0 = 2 * j
            i1 = 2 * j + 1
            i2 = jnp.minimum(i0 + 2, last)
            pltpu.async_copy(y_hbm.at[s_at(i1)], rows1, semg1)
            pltpu.make_async_copy(y_hbm.at[s_at(i0)], rows0, semg0).wait()
            pltpu.sync_copy(rows0, acc.at[d_at(i0)], add=True)
            pltpu.async_copy(y_hbm.at[s_at(i2)], rows0, semg0)
            pltpu.make_async_copy(y_hbm.at[s_at(i1)], rows1, semg1).wait()
            pltpu.sync_copy(rows1, acc.at[d_at(i1)], add=True)
            return carry

        lax.fori_loop(0, n_iters // 2, body, 0)
        pltpu.make_async_copy(y_hbm.at[s_at(last)], rows0, semg0).wait()
        plsc.subcore_barrier()
        pltpu.sync_copy(acc.at[pl.ds(sid * RPT, RPT)], out_hbm.at[cid, sid])

    return agg_kernel


_BR = 1000


def _tc_first(x, w, dinv2):
    n, d_in = x.shape
    d_out = w.shape[1]

    def body(x_ref, w_ref, dinv_ref, y_ref):
        xw = jnp.dot(x_ref[...], w_ref[...],
                     preferred_element_type=jnp.float32)
        y_ref[...] = xw * dinv_ref[...]

    return pl.pallas_call(
        body,
        grid=(n // _BR,),
        in_specs=[
            pl.BlockSpec((_BR, d_in), lambda i: (i, 0)),
            pl.BlockSpec((d_in, d_out), lambda i: (0, 0)),
            pl.BlockSpec((_BR, 1), lambda i: (i, 0)),
        ],
        out_specs=pl.BlockSpec((_BR, d_out), lambda i: (i, 0)),
        out_shape=jax.ShapeDtypeStruct((n, d_out), jnp.float32),
    )(x, w, dinv2)


def _tc_mid(p0, p1, y_prev, dinv2, b, w):
    n, d = y_prev.shape
    d_out = w.shape[1]

    def body(p0_ref, p1_ref, y_ref, dinv_ref, b_ref, w_ref, out_ref):
        agg = p0_ref[...] + p1_ref[...] + y_ref[...]
        h = jnp.maximum(agg * dinv_ref[...] + b_ref[...], 0.0)
        hw = jnp.dot(h, w_ref[...], preferred_element_type=jnp.float32)
        out_ref[...] = hw * dinv_ref[...]

    return pl.pallas_call(
        body,
        grid=(n // _BR,),
        in_specs=[
            pl.BlockSpec((_BR, d), lambda i: (i, 0)),
            pl.BlockSpec((_BR, d), lambda i: (i, 0)),
            pl.BlockSpec((_BR, d), lambda i: (i, 0)),
            pl.BlockSpec((_BR, 1), lambda i: (i, 0)),
            pl.BlockSpec((1, d), lambda i: (0, 0)),
            pl.BlockSpec((d, d_out), lambda i: (0, 0)),
        ],
        out_specs=pl.BlockSpec((_BR, d_out), lambda i: (i, 0)),
        out_shape=jax.ShapeDtypeStruct((n, d_out), jnp.float32),
    )(p0, p1, y_prev, dinv2, b, w)


def _tc_scale(p0, p1, y_prev, dinv2, b):
    n, d = y_prev.shape

    def body(p0_ref, p1_ref, y_ref, dinv_ref, b_ref, out_ref):
        agg = p0_ref[...] + p1_ref[...] + y_ref[...]
        h = jnp.maximum(agg * dinv_ref[...] + b_ref[...], 0.0)
        out_ref[...] = h * dinv_ref[...]

    return pl.pallas_call(
        body,
        grid=(n // _BR,),
        in_specs=[
            pl.BlockSpec((_BR, d), lambda i: (i, 0)),
            pl.BlockSpec((_BR, d), lambda i: (i, 0)),
            pl.BlockSpec((_BR, d), lambda i: (i, 0)),
            pl.BlockSpec((_BR, 1), lambda i: (i, 0)),
            pl.BlockSpec((1, d), lambda i: (0, 0)),
        ],
        out_specs=pl.BlockSpec((_BR, d), lambda i: (i, 0)),
        out_shape=jax.ShapeDtypeStruct((n, d), jnp.float32),
    )(p0, p1, y_prev, dinv2, b)


def _tc_final(p0, p1, t_prev, dinv2, w, b):
    n, d = t_prev.shape
    d_out = w.shape[1]

    def body(p0_ref, p1_ref, t_ref, dinv_ref, w_ref, b_ref, out_ref):
        agg = (p0_ref[...] + p1_ref[...] + t_ref[...]) * dinv_ref[...]
        z = jnp.dot(agg, w_ref[...],
                    preferred_element_type=jnp.float32) + b_ref[...]
        m = jnp.max(z, axis=1, keepdims=True)
        e = jnp.exp(z - m)
        s = jnp.sum(e, axis=1, keepdims=True)
        out_ref[...] = z - m - jnp.log(s)

    return pl.pallas_call(
        body,
        grid=(n // _BR,),
        in_specs=[
            pl.BlockSpec((_BR, d), lambda i: (i, 0)),
            pl.BlockSpec((_BR, d), lambda i: (i, 0)),
            pl.BlockSpec((_BR, d), lambda i: (i, 0)),
            pl.BlockSpec((_BR, 1), lambda i: (i, 0)),
            pl.BlockSpec((d, d_out), lambda i: (0, 0)),
            pl.BlockSpec((1, d_out), lambda i: (0, 0)),
        ],
        out_specs=pl.BlockSpec((_BR, d_out), lambda i: (i, 0)),
        out_shape=jax.ShapeDtypeStruct((n, d_out), jnp.float32),
    )(p0, p1, t_prev, dinv2, w, b)


def kernel(x, edge_index, W_in, b_in, W_h, b_h, W_out, b_out):
    n, d_in = x.shape
    n_edges = edge_index.shape[1]
    d_h = W_h.shape[0]
    d_out = W_out.shape[1]
    n_pad = NS * RPT

    ei = edge_index.astype(jnp.int32)
    e_blk = 2 * CHUNK * NW
    e_pad = ((n_edges + e_blk - 1) // e_blk) * e_blk
    src = jnp.concatenate(
        [ei[0], jnp.zeros((e_pad - n_edges,), jnp.int32)])
    dst = jnp.concatenate(
        [ei[1], jnp.full((e_pad - n_edges,), n_pad - 1, jnp.int32)])
    degp = _make_deg_kernel(e_pad)(dst)
    degp = degp.reshape(NC, n_pad, 128)
    deg = degp[0, :n, 0] + degp[1, :n, 0] + 1.0
    dinv2 = lax.rsqrt(deg)[:, None]

    agg_h = _make_agg_kernel(e_pad, d_h)

    y1 = _tc_first(x, W_in, dinv2)
    p = agg_h(y1, src, dst).reshape(NC, n_pad, d_h)[:, :n]
    y2 = _tc_mid(p[0], p[1], y1, dinv2, b_in.reshape(1, -1), W_h)
    p = agg_h(y2, src, dst).reshape(NC, n_pad, d_h)[:, :n]
    t3 = _tc_scale(p[0], p[1], y2, dinv2, b_h.reshape(1, -1))
    p = agg_h(t3, src, dst).reshape(NC, n_pad, d_h)[:, :n]
    return _tc_final(p[0], p[1], t3, dinv2, W_out, b_out.reshape(1, -1))
